# baseline (speedup 1.0000x reference)
import functools

import jax
import jax.numpy as jnp
from jax import lax
from jax.experimental import pallas as pl
from jax.experimental.pallas import tpu as pltpu
from jax.experimental.pallas import tpu_sc as plsc

N = 2_000_000
SIGMA = 9.0
T = 1.0 / SIGMA
HALF_SIGMA = 0.5 * SIGMA

NC = 2
NS = 16
NW = NC * NS
L = 16

CH = 2000
NCHUNKS = N // CH
GROUPS = CH // L


def _sc_body(inp_hbm, tgt_hbm, out_hbm, tbuf, ibuf, stage):
    wid = lax.axis_index("s") * NC + lax.axis_index("c")
    nk = (NCHUNKS + NW - 1 - wid) // NW

    iota = lax.iota(jnp.int32, L)
    iota3 = iota * 3
    iota2 = iota * 2

    def chunk_body(k, carry):
        acc, cnt = carry
        c = wid + k * NW
        pltpu.sync_copy(tgt_hbm.at[pl.ds(c * (CH * 3), CH * 3)], tbuf)
        pltpu.sync_copy(inp_hbm.at[pl.ds(c * (CH * 2), CH * 2)], ibuf)

        def group_body(g, carry2):
            acc2, cnt2 = carry2
            b3 = g * (3 * L)
            b2 = g * (2 * L)
            i0 = iota3 + b3
            j0 = iota2 + b2
            cls = plsc.load_gather(tbuf, [i0])
            r0 = plsc.load_gather(tbuf, [i0 + 1])
            r1 = plsc.load_gather(tbuf, [i0 + 2])
            p0 = plsc.load_gather(ibuf, [j0])
            p1 = plsc.load_gather(ibuf, [j0 + 1])
            d0 = jnp.abs(r0 - p0)
            d1 = jnp.abs(r1 - p1)
            m0 = jnp.minimum(d0, T)
            m1 = jnp.minimum(d1, T)
            l = (d0 - m0) + (d1 - m1) + HALF_SIGMA * (m0 * m0 + m1 * m1)
            return acc2 + cls * l, cnt2 + cls

        return lax.fori_loop(0, GROUPS, group_body, (acc, cnt))

    zero = jnp.zeros((L,), jnp.float32)
    acc, cnt = lax.fori_loop(0, nk, chunk_body, (zero, zero))
    stage[pl.ds(0, L)] = acc
    stage[pl.ds(L, L)] = cnt
    pltpu.sync_copy(stage, out_hbm.at[pl.ds(wid * (2 * L), 2 * L)])


@jax.jit
def _rpn_regr_loss(inp_flat, tgt_flat):
    mesh = plsc.VectorSubcoreMesh(core_axis_name="c", subcore_axis_name="s")
    partials = pl.kernel(
        _sc_body,
        out_type=jax.ShapeDtypeStruct((NW * 2 * L,), jnp.float32),
        mesh=mesh,
        scratch_types=[
            pltpu.VMEM((CH * 3,), jnp.float32),
            pltpu.VMEM((CH * 2,), jnp.float32),
            pltpu.VMEM((2 * L,), jnp.float32),
        ],
        compiler_params=pltpu.CompilerParams(needs_layout_passes=False),
    )(inp_flat, tgt_flat)
    p = partials.reshape(NW, 2, L)
    total = jnp.sum(p[:, 0, :])
    cnt = jnp.sum(p[:, 1, :])
    return jnp.where(cnt > 0, total / jnp.maximum(cnt, 1.0), 0.0)


def kernel(input_data, target):
    inp_flat = input_data.astype(jnp.float32).reshape(-1)
    tgt_flat = target.astype(jnp.float32).reshape(-1)
    return _rpn_regr_loss(inp_flat, tgt_flat)


# speedup vs baseline: 87.2379x; 87.2379x over previous
import jax
import jax.numpy as jnp
from jax import lax
from jax.experimental import pallas as pl
from jax.experimental.pallas import tpu as pltpu
from jax.experimental.pallas import tpu_sc as plsc

N = 2_000_000
SIGMA = 9.0
T = 1.0 / SIGMA
HALF_SIGMA = 0.5 * SIGMA

NC = 2
NS = 16
NW = NC * NS
L = 16

BLK = 128
NBLK = N // BLK
B = 25
CH = B * BLK
NCHUNKS = NBLK // B


def _sc_body(tgt_hbm, ip_hbm, out_hbm, cbuf, r0buf, r1buf, ibuf, stage):
    wid = lax.axis_index("s") * NC + lax.axis_index("c")
    nk = (NCHUNKS + NW - 1 - wid) // NW

    def chunk_body(k, carry):
        acc, cnt = carry
        bb = (wid + k * NW) * B
        pltpu.sync_copy(tgt_hbm.at[pl.ds(bb, B), :, :], cbuf)
        pltpu.sync_copy(tgt_hbm.at[pl.ds(NBLK + bb, B), :, :], r0buf)
        pltpu.sync_copy(tgt_hbm.at[pl.ds(2 * NBLK + bb, B), :, :], r1buf)
        pltpu.sync_copy(ip_hbm.at[pl.ds(bb, B), :, :], ibuf)

        def block_body(b, carry2):
            acc2, cnt2 = carry2
            for j in range(BLK // L):
                s = pl.ds(j * L, L)
                cls = cbuf[b, 0, s]
                r0 = r0buf[b, 0, s]
                r1 = r1buf[b, 0, s]
                p0 = ibuf[b, 0, s]
                p1 = ibuf[b, 1, s]
                d0 = jnp.abs(r0 - p0)
                d1 = jnp.abs(r1 - p1)
                m0 = jnp.minimum(d0, T)
                m1 = jnp.minimum(d1, T)
                l = (d0 - m0) + (d1 - m1) + HALF_SIGMA * (m0 * m0 + m1 * m1)
                acc2 = acc2 + cls * l
                cnt2 = cnt2 + cls
            return acc2, cnt2

        return lax.fori_loop(0, B, block_body, (acc, cnt))

    zero = jnp.zeros((L,), jnp.float32)
    acc, cnt = lax.fori_loop(0, nk, chunk_body, (zero, zero))
    stage[pl.ds(0, L)] = acc
    stage[pl.ds(L, L)] = cnt
    pltpu.sync_copy(stage, out_hbm.at[pl.ds(wid * (2 * L), 2 * L)])


@jax.jit
def _rpn_regr_loss(input_data, target):
    tgt_v = lax.reshape(target, (3 * NBLK, 1, BLK), dimensions=(0, 2, 1))
    ip_v = input_data[0].reshape(NBLK, BLK, 2).transpose(0, 2, 1)

    mesh = plsc.VectorSubcoreMesh(core_axis_name="c", subcore_axis_name="s")
    partials = pl.kernel(
        _sc_body,
        out_type=jax.ShapeDtypeStruct((NW * 2 * L,), jnp.float32),
        mesh=mesh,
        scratch_types=[
            pltpu.VMEM((B, 1, BLK), jnp.float32),
            pltpu.VMEM((B, 1, BLK), jnp.float32),
            pltpu.VMEM((B, 1, BLK), jnp.float32),
            pltpu.VMEM((B, 2, BLK), jnp.float32),
            pltpu.VMEM((2 * L,), jnp.float32),
        ],
        compiler_params=pltpu.CompilerParams(needs_layout_passes=False),
    )(tgt_v, ip_v)
    p = partials.reshape(NW, 2, L)
    total = jnp.sum(p[:, 0, :])
    cnt = jnp.sum(p[:, 1, :])
    return jnp.where(cnt > 0, total / jnp.maximum(cnt, 1.0), 0.0)


def kernel(input_data, target):
    return _rpn_regr_loss(input_data.astype(jnp.float32),
                          target.astype(jnp.float32))


# speedup vs baseline: 184.5833x; 2.1159x over previous
import jax
import jax.numpy as jnp
from jax import lax
from jax.experimental import pallas as pl
from jax.experimental.pallas import tpu as pltpu
from jax.experimental.pallas import tpu_sc as plsc

N = 2_000_000
SIGMA = 9.0
T = 1.0 / SIGMA
HALF_SIGMA = 0.5 * SIGMA

NC = 2
NS = 16
NW = NC * NS
L = 16

BLK = 128
NBLK = N // BLK
B = 25
CH = B * BLK
NCHUNKS = NBLK // B
MAXK = (NCHUNKS + NW - 1) // NW
PAIRS = (MAXK - 2) // 2


def _sc_body(tgt_hbm, ip_hbm, out_hbm,
             cb0, r0b0, r1b0, ib0, cb1, r0b1, r1b1, ib1,
             stage, sem0, sem1):
    wid = lax.axis_index("s") * NC + lax.axis_index("c")
    nk = (NCHUNKS + NW - 1 - wid) // NW

    def start(m, cb, r0b, r1b, ib, sem):
        bb = (wid + m * NW) * B
        pltpu.async_copy(tgt_hbm.at[pl.ds(bb, B), :, :], cb, sem)
        pltpu.async_copy(tgt_hbm.at[pl.ds(NBLK + bb, B), :, :], r0b, sem)
        pltpu.async_copy(tgt_hbm.at[pl.ds(2 * NBLK + bb, B), :, :], r1b, sem)
        pltpu.async_copy(ip_hbm.at[pl.ds(bb, B), :, :], ib, sem)

    def wait(cb, r0b, r1b, ib, sem):
        pltpu.make_async_copy(tgt_hbm.at[pl.ds(0, B), :, :], cb, sem).wait()
        pltpu.make_async_copy(tgt_hbm.at[pl.ds(0, B), :, :], r0b, sem).wait()
        pltpu.make_async_copy(tgt_hbm.at[pl.ds(0, B), :, :], r1b, sem).wait()
        pltpu.make_async_copy(ip_hbm.at[pl.ds(0, B), :, :], ib, sem).wait()

    def compute(cb, r0b, r1b, ib, acc, cnt):
        def block_body(b, carry2):
            acc2, cnt2 = carry2
            for j in range(BLK // L):
                s = pl.ds(j * L, L)
                cls = cb[b, 0, s]
                r0 = r0b[b, 0, s]
                r1 = r1b[b, 0, s]
                p0 = ib[b, 0, s]
                p1 = ib[b, 1, s]
                d0 = jnp.abs(r0 - p0)
                d1 = jnp.abs(r1 - p1)
                m0 = jnp.minimum(d0, T)
                m1 = jnp.minimum(d1, T)
                l = (d0 - m0) + (d1 - m1) + HALF_SIGMA * (m0 * m0 + m1 * m1)
                acc2 = acc2 + cls * l
                cnt2 = cnt2 + cls
            return acc2, cnt2

        return lax.fori_loop(0, B, block_body, (acc, cnt))

    start(0, cb0, r0b0, r1b0, ib0, sem0)
    start(1, cb1, r0b1, r1b1, ib1, sem1)

    def pair_body(t2, carry):
        acc, cnt = carry
        m0 = 2 * t2
        wait(cb0, r0b0, r1b0, ib0, sem0)
        acc, cnt = compute(cb0, r0b0, r1b0, ib0, acc, cnt)
        start(m0 + 2, cb0, r0b0, r1b0, ib0, sem0)

        wait(cb1, r0b1, r1b1, ib1, sem1)
        acc, cnt = compute(cb1, r0b1, r1b1, ib1, acc, cnt)

        @pl.when(m0 + 3 < nk)
        def _():
            start(m0 + 3, cb1, r0b1, r1b1, ib1, sem1)

        return acc, cnt

    zero = jnp.zeros((L,), jnp.float32)
    acc, cnt = lax.fori_loop(0, PAIRS, pair_body, (zero, zero))

    wait(cb0, r0b0, r1b0, ib0, sem0)
    acc, cnt = compute(cb0, r0b0, r1b0, ib0, acc, cnt)

    def tail(carry):
        a, c = carry
        wait(cb1, r0b1, r1b1, ib1, sem1)
        return compute(cb1, r0b1, r1b1, ib1, a, c)

    acc, cnt = lax.cond(2 * PAIRS + 1 < nk, tail, lambda carry: carry, (acc, cnt))

    stage[pl.ds(0, L)] = acc
    stage[pl.ds(L, L)] = cnt
    pltpu.sync_copy(stage, out_hbm.at[pl.ds(wid * (2 * L), 2 * L)])


@jax.jit
def _rpn_regr_loss(input_data, target):
    tgt_v = lax.reshape(target, (3 * NBLK, 1, BLK), dimensions=(0, 2, 1))
    ip_v = input_data[0].reshape(NBLK, BLK, 2).transpose(0, 2, 1)

    mesh = plsc.VectorSubcoreMesh(core_axis_name="c", subcore_axis_name="s")
    partials = pl.kernel(
        _sc_body,
        out_type=jax.ShapeDtypeStruct((NW * 2 * L,), jnp.float32),
        mesh=mesh,
        scratch_types=[
            pltpu.VMEM((B, 1, BLK), jnp.float32),
            pltpu.VMEM((B, 1, BLK), jnp.float32),
            pltpu.VMEM((B, 1, BLK), jnp.float32),
            pltpu.VMEM((B, 2, BLK), jnp.float32),
            pltpu.VMEM((B, 1, BLK), jnp.float32),
            pltpu.VMEM((B, 1, BLK), jnp.float32),
            pltpu.VMEM((B, 1, BLK), jnp.float32),
            pltpu.VMEM((B, 2, BLK), jnp.float32),
            pltpu.VMEM((2 * L,), jnp.float32),
            pltpu.SemaphoreType.DMA,
            pltpu.SemaphoreType.DMA,
        ],
        compiler_params=pltpu.CompilerParams(needs_layout_passes=False),
    )(tgt_v, ip_v)
    p = partials.reshape(NW, 2, L)
    total = jnp.sum(p[:, 0, :])
    cnt = jnp.sum(p[:, 1, :])
    return jnp.where(cnt > 0, total / jnp.maximum(cnt, 1.0), 0.0)


def kernel(input_data, target):
    return _rpn_regr_loss(input_data.astype(jnp.float32),
                          target.astype(jnp.float32))
